# baseline (speedup 1.0000x reference)
import jax
import jax.numpy as jnp
from jax.experimental import pallas as pl
from jax.experimental.pallas import tpu as pltpu

_N = 16384
_C_IN = 128
_C_OUT = 16
_EPS = 1e-5
_BM = 384
_NWHOLE = 42
_NSUB = 2
_BT = 128
_ZCH = 1024
_NZC = _N // _ZCH


def _block_update(blk, zconvT_ref, col0, sums, sumsq):
    blkT = blk.T
    zconvT_ref[:, pl.ds(col0, blkT.shape[1])] = blkT
    s = jnp.sum(blkT, axis=1, keepdims=True)
    sq = jnp.sum(blkT * blkT, axis=1, keepdims=True)
    return sums + s, sumsq + sq


def _simplicial_conv_kernel(zh_hbm, l1_hbm, w_ref, b_ref, g_ref, beta_ref,
                            out_ref, zh_vmem, zthetaT_ref, zconvT_ref,
                            l1_buf, zh_sems, l1_sems, tail_sems):
    for c in range(2):
        pltpu.make_async_copy(
            zh_hbm.at[pl.ds(c * _ZCH, _ZCH), :], zh_vmem.at[c],
            zh_sems.at[c]).start()
    for s in range(2):
        pltpu.make_async_copy(
            l1_hbm.at[pl.ds(s * _BM, _BM), :], l1_buf.at[s], l1_sems.at[s]
        ).start()
    w = w_ref[...]
    b_col = b_ref[...]
    for c in range(_NZC):
        pltpu.make_async_copy(
            zh_hbm.at[pl.ds(c * _ZCH, _ZCH), :], zh_vmem.at[c % 2],
            zh_sems.at[c % 2]).wait()
        zthetaT_ref[:, pl.ds(c * _ZCH, _ZCH)] = (
            jax.lax.dot_general(
                w, zh_vmem[c % 2], (((1,), (1,)), ((), ())),
                preferred_element_type=jnp.float32)
            + b_col
        )
        if c + 2 < _NZC:
            pltpu.make_async_copy(
                zh_hbm.at[pl.ds((c + 2) * _ZCH, _ZCH), :],
                zh_vmem.at[c % 2], zh_sems.at[c % 2]).start()

    zeros = jnp.zeros((_C_OUT, 1), jnp.float32)

    def body(k, carry):
        sums, sumsq = carry
        slot = jax.lax.rem(k, 2)
        pltpu.make_async_copy(
            l1_hbm.at[pl.ds(k * _BM, _BM), :], l1_buf.at[slot],
            l1_sems.at[slot]
        ).wait()
        blk = jax.lax.dot_general(
            l1_buf[slot], zthetaT_ref[...], (((1,), (1,)), ((), ())),
            preferred_element_type=jnp.float32)
        sums, sumsq = _block_update(blk, zconvT_ref, k * _BM, sums, sumsq)
        nk = k + 2

        @pl.when(nk < _NWHOLE)
        def _prefetch_whole():
            pltpu.make_async_copy(
                l1_hbm.at[pl.ds(nk * _BM, _BM), :], l1_buf.at[slot],
                l1_sems.at[slot]
            ).start()

        @pl.when(nk == _NWHOLE)
        def _prefetch_tail():
            for t in range(_NSUB):
                pltpu.make_async_copy(
                    l1_hbm.at[pl.ds(nk * _BM + t * _BT, _BT), :],
                    l1_buf.at[_NWHOLE % 2].at[pl.ds(t * _BT, _BT), :],
                    tail_sems.at[t],
                ).start()

        return sums, sumsq

    sums, sumsq = jax.lax.fori_loop(0, _NWHOLE, body, (zeros, zeros))

    tail_slot = _NWHOLE % 2
    for t in range(_NSUB):
        pltpu.make_async_copy(
            l1_hbm.at[pl.ds(_NWHOLE * _BM + t * _BT, _BT), :],
            l1_buf.at[tail_slot].at[pl.ds(t * _BT, _BT), :],
            tail_sems.at[t],
        ).wait()
        blk = jax.lax.dot_general(
            l1_buf[tail_slot, t * _BT:(t + 1) * _BT, :], zthetaT_ref[...],
            (((1,), (1,)), ((), ())), preferred_element_type=jnp.float32)
        sums, sumsq = _block_update(
            blk, zconvT_ref, _NWHOLE * _BM + t * _BT, sums, sumsq)

    inv_n = jnp.float32(1.0 / _N)
    mean = sums * inv_n
    var = sumsq * inv_n - mean * mean
    inv = jax.lax.rsqrt(var + _EPS)
    scale = g_ref[...] * inv
    shift = beta_ref[...] - mean * scale
    zp = jnp.maximum(scale * zconvT_ref[...] + shift, 0.0)
    out_ref[...] = jnp.max(zp, axis=0, keepdims=True)


def kernel(Z_H, L1_tilde, W, b, gamma, beta):
    b_col = b.reshape(_C_OUT, 1)
    g_col = gamma.reshape(_C_OUT, 1)
    beta_col = beta.reshape(_C_OUT, 1)

    out = pl.pallas_call(
        _simplicial_conv_kernel,
        in_specs=[
            pl.BlockSpec(memory_space=pl.ANY),
            pl.BlockSpec(memory_space=pl.ANY),
            pl.BlockSpec((_C_OUT, _C_IN), lambda: (0, 0)),
            pl.BlockSpec((_C_OUT, 1), lambda: (0, 0)),
            pl.BlockSpec((_C_OUT, 1), lambda: (0, 0)),
            pl.BlockSpec((_C_OUT, 1), lambda: (0, 0)),
        ],
        out_specs=pl.BlockSpec((1, _N), lambda: (0, 0)),
        out_shape=jax.ShapeDtypeStruct((1, _N), jnp.float32),
        scratch_shapes=[
            pltpu.VMEM((2, _ZCH, _C_IN), jnp.float32),
            pltpu.VMEM((_C_OUT, _N), jnp.float32),
            pltpu.VMEM((_C_OUT, _N), jnp.float32),
            pltpu.VMEM((2, _BM, _N), jnp.float32),
            pltpu.SemaphoreType.DMA((2,)),
            pltpu.SemaphoreType.DMA((2,)),
            pltpu.SemaphoreType.DMA((_NSUB,)),
        ],
    )(Z_H, L1_tilde, W, b_col, g_col, beta_col)
    return out.reshape(_N, 1)


# speedup vs baseline: 1.0410x; 1.0410x over previous
import jax
import jax.numpy as jnp
from jax.experimental import pallas as pl
from jax.experimental.pallas import tpu as pltpu

_N = 16384
_C_IN = 128
_C_OUT = 16
_EPS = 1e-5
_BM = 256
_NB = _N // _BM
_NSUB = 4
_BT = _BM // _NSUB


def _block_update(blk, zconvT_ref, col0, sums, sumsq):
    blkT = blk.T
    zconvT_ref[:, pl.ds(col0, blkT.shape[1])] = blkT
    s = jnp.sum(blkT, axis=1, keepdims=True)
    sq = jnp.sum(blkT * blkT, axis=1, keepdims=True)
    return sums + s, sumsq + sq


def _simplicial_conv_kernel(zh_hbm, l1_hbm, w_ref, b_ref, g_ref, beta_ref,
                            out_ref, zh_vmem, ztheta_ref, zconvT_ref,
                            l1_buf, zh_sem, l1_sems, tail_sems):
    zh_copy = pltpu.make_async_copy(zh_hbm, zh_vmem, zh_sem)
    zh_copy.start()
    for s in range(2):
        pltpu.make_async_copy(
            l1_hbm.at[pl.ds(s * _BM, _BM), :], l1_buf.at[s], l1_sems.at[s]
        ).start()
    zh_copy.wait()
    ztheta_ref[...] = (
        jnp.dot(zh_vmem[...], w_ref[...].T,
                preferred_element_type=jnp.float32)
        + b_ref[...]
    )

    zeros = jnp.zeros((_C_OUT, 1), jnp.float32)

    def body(k, carry):
        sums, sumsq = carry
        slot = jax.lax.rem(k, 2)
        pltpu.make_async_copy(
            l1_hbm.at[pl.ds(k * _BM, _BM), :], l1_buf.at[slot],
            l1_sems.at[slot]
        ).wait()
        blk = jnp.dot(l1_buf[slot], ztheta_ref[...],
                      preferred_element_type=jnp.float32)
        sums, sumsq = _block_update(blk, zconvT_ref, k * _BM, sums, sumsq)
        nk = k + 2

        @pl.when(nk < _NB - 1)
        def _prefetch_whole():
            pltpu.make_async_copy(
                l1_hbm.at[pl.ds(nk * _BM, _BM), :], l1_buf.at[slot],
                l1_sems.at[slot]
            ).start()

        @pl.when(nk == _NB - 1)
        def _prefetch_tail():
            for t in range(_NSUB):
                pltpu.make_async_copy(
                    l1_hbm.at[pl.ds(nk * _BM + t * _BT, _BT), :],
                    l1_buf.at[(_NB - 1) % 2].at[pl.ds(t * _BT, _BT), :],
                    tail_sems.at[t],
                ).start()

        return sums, sumsq

    sums, sumsq = jax.lax.fori_loop(0, _NB - 1, body, (zeros, zeros))

    tail_slot = (_NB - 1) % 2
    for t in range(_NSUB):
        pltpu.make_async_copy(
            l1_hbm.at[pl.ds((_NB - 1) * _BM + t * _BT, _BT), :],
            l1_buf.at[tail_slot].at[pl.ds(t * _BT, _BT), :],
            tail_sems.at[t],
        ).wait()
        blk = jnp.dot(l1_buf[tail_slot, t * _BT:(t + 1) * _BT, :],
                      ztheta_ref[...], preferred_element_type=jnp.float32)
        sums, sumsq = _block_update(
            blk, zconvT_ref, (_NB - 1) * _BM + t * _BT, sums, sumsq)

    inv_n = jnp.float32(1.0 / _N)
    mean = sums * inv_n
    var = sumsq * inv_n - mean * mean
    inv = jax.lax.rsqrt(var + _EPS)
    scale = g_ref[...] * inv
    shift = beta_ref[...] - mean * scale
    zp = jnp.maximum(scale * zconvT_ref[...] + shift, 0.0)
    out_ref[...] = jnp.max(zp, axis=0, keepdims=True)


def kernel(Z_H, L1_tilde, W, b, gamma, beta):
    b_row = b.reshape(1, _C_OUT)
    g_col = gamma.reshape(_C_OUT, 1)
    beta_col = beta.reshape(_C_OUT, 1)

    out = pl.pallas_call(
        _simplicial_conv_kernel,
        in_specs=[
            pl.BlockSpec(memory_space=pl.ANY),
            pl.BlockSpec(memory_space=pl.ANY),
            pl.BlockSpec((_C_OUT, _C_IN), lambda: (0, 0)),
            pl.BlockSpec((1, _C_OUT), lambda: (0, 0)),
            pl.BlockSpec((_C_OUT, 1), lambda: (0, 0)),
            pl.BlockSpec((_C_OUT, 1), lambda: (0, 0)),
        ],
        out_specs=pl.BlockSpec((1, _N), lambda: (0, 0)),
        out_shape=jax.ShapeDtypeStruct((1, _N), jnp.float32),
        scratch_shapes=[
            pltpu.VMEM((_N, _C_IN), jnp.float32),
            pltpu.VMEM((_N, _C_OUT), jnp.float32),
            pltpu.VMEM((_C_OUT, _N), jnp.float32),
            pltpu.VMEM((2, _BM, _N), jnp.float32),
            pltpu.SemaphoreType.DMA,
            pltpu.SemaphoreType.DMA((2,)),
            pltpu.SemaphoreType.DMA((_NSUB,)),
        ],
    )(Z_H, L1_tilde, W, b_row, g_col, beta_col)
    return out.reshape(_N, 1)
